# Initial kernel scaffold; baseline (speedup 1.0000x reference)
import jax
import jax.numpy as jnp
from jax.experimental import pallas as pl


def kernel(src, index):
    raise NotImplementedError("write your pallas kernel here")


# speedup vs baseline: 4.5908x; 4.5908x over previous
import functools

import jax
import jax.numpy as jnp
from jax import lax
from jax.experimental import pallas as pl
from jax.experimental.pallas import tpu as pltpu
from jax.experimental.pallas import tpu_sc as plsc

N = 10000
E = 320000
D = 128

NC = 2
NS = 16
NW = NC * NS

ROWS_PER_TILE = E // NW
CHUNK = 80
NCHUNK = ROWS_PER_TILE // CHUNK
WIN = 640
WIN_STRIDE = 624


def _sc_partials(src, idx3):
    mesh = plsc.VectorSubcoreMesh(core_axis_name="c", subcore_axis_name="s")

    @functools.partial(
        pl.kernel,
        mesh=mesh,
        out_type=jax.ShapeDtypeStruct((NC, N, D), jnp.float32),
        scratch_types=[
            pltpu.VMEM_SHARED((N, D), jnp.float32),
            pltpu.VMEM((NCHUNK, CHUNK), jnp.int32),
            pltpu.VMEM((CHUNK, D), jnp.float32),
        ],
    )
    def body(src_hbm, idx_hbm, out_hbm, acc, idx_v, buf):
        c = lax.axis_index("c")
        s = lax.axis_index("s")
        wid = c * NS + s
        row0 = wid * ROWS_PER_TILE
        win0 = pl.multiple_of(s * WIN_STRIDE, 8)

        zeros16 = jnp.zeros((16,), jnp.float32)

        def zero_row(r, _):
            for k in range(D // 16):
                buf[r, pl.ds(k * 16, 16)] = zeros16
            return 0

        lax.fori_loop(0, CHUNK, zero_row, 0)
        for z in range(WIN // CHUNK):
            pltpu.sync_copy(
                buf, acc.at[pl.ds(pl.multiple_of(win0 + z * CHUNK, 8), CHUNK)]
            )
        plsc.subcore_barrier()

        pltpu.sync_copy(idx_hbm.at[wid], idx_v)

        def chunk_step(j, _):
            src_off = pl.multiple_of(row0 + j * CHUNK, 8)
            pltpu.sync_copy(src_hbm.at[pl.ds(src_off, CHUNK)], buf)
            pltpu.sync_copy(buf, acc.at[idx_v.at[j]], add=True)
            return 0

        lax.fori_loop(0, NCHUNK, chunk_step, 0)
        plsc.subcore_barrier()

        pltpu.sync_copy(
            acc.at[pl.ds(win0, WIN)],
            out_hbm.at[c].at[pl.ds(win0, WIN)],
        )

    return body(src, idx3)


def _combine(partials):
    def body(p_ref, o_ref):
        o_ref[...] = p_ref[0] + p_ref[1]

    blk = 1000
    return pl.pallas_call(
        body,
        grid=(N // blk,),
        in_specs=[pl.BlockSpec((NC, blk, D), lambda i: (0, i, 0))],
        out_specs=pl.BlockSpec((blk, D), lambda i: (i, 0)),
        out_shape=jax.ShapeDtypeStruct((N, D), jnp.float32),
    )(partials)


def kernel(src, index):
    idx3 = index.reshape(NW, NCHUNK, CHUNK)
    partials = _sc_partials(src, idx3)
    return _combine(partials)


# speedup vs baseline: 5.9259x; 1.2908x over previous
import functools

import jax
import jax.numpy as jnp
from jax import lax
from jax.experimental import pallas as pl
from jax.experimental.pallas import tpu as pltpu
from jax.experimental.pallas import tpu_sc as plsc

N = 10000
E = 320000
D = 128

NC = 2
NS = 16
NW = NC * NS

ROWS_PER_TILE = E // NW
CHUNK = 80
NCHUNK = ROWS_PER_TILE // CHUNK
FILL = 80
NFILL = ROWS_PER_TILE // FILL
SPF = FILL // CHUNK
WIN = 640
WIN_STRIDE = 624


def _sc_partials(src, idx3):
    mesh = plsc.VectorSubcoreMesh(core_axis_name="c", subcore_axis_name="s")

    @functools.partial(
        pl.kernel,
        mesh=mesh,
        out_type=jax.ShapeDtypeStruct((NC, N, D), jnp.float32),
        scratch_types=[
            pltpu.VMEM_SHARED((N, D), jnp.float32),
            pltpu.VMEM((NCHUNK, CHUNK), jnp.int32),
            pltpu.VMEM((FILL, D), jnp.float32),
            pltpu.VMEM((FILL, D), jnp.float32),
            pltpu.SemaphoreType.DMA,
            pltpu.SemaphoreType.DMA,
        ],
    )
    def body(src_hbm, idx_hbm, out_hbm, acc, idx_v, buf0, buf1, sem0, sem1):
        c = lax.axis_index("c")
        s = lax.axis_index("s")
        wid = c * NS + s
        row0 = wid * ROWS_PER_TILE
        win0 = pl.multiple_of(s * WIN_STRIDE, 8)

        zeros16 = jnp.zeros((16,), jnp.float32)

        def zero_row(r, _):
            for k in range(D // 16):
                buf0[r, pl.ds(k * 16, 16)] = zeros16
            return 0

        lax.fori_loop(0, FILL, zero_row, 0)
        for z in range(WIN // FILL):
            pltpu.sync_copy(
                buf0, acc.at[pl.ds(pl.multiple_of(win0 + z * FILL, 8), FILL)]
            )
        plsc.subcore_barrier()

        pltpu.sync_copy(idx_hbm.at[wid], idx_v)

        def fill_start(f, buf, sem):
            src_off = pl.multiple_of(row0 + f * FILL, 8)
            pltpu.async_copy(src_hbm.at[pl.ds(src_off, FILL)], buf, sem)

        def fill_wait(buf, sem):
            pltpu.make_async_copy(src_hbm.at[pl.ds(0, FILL)], buf, sem).wait()

        def drain(f, buf):
            for k in range(SPF):
                pltpu.sync_copy(
                    buf.at[pl.ds(k * CHUNK, CHUNK)],
                    acc.at[idx_v.at[f * SPF + k]],
                    add=True,
                )

        fill_start(0, buf0, sem0)

        def pair_step(g, _):
            fill_wait(buf0, sem0)
            fill_start(2 * g + 1, buf1, sem1)
            drain(2 * g, buf0)
            fill_wait(buf1, sem1)
            fill_start(2 * g + 2, buf0, sem0)
            drain(2 * g + 1, buf1)
            return 0

        lax.fori_loop(0, (NFILL - 1) // 2, pair_step, 0)
        fill_wait(buf0, sem0)
        drain(NFILL - 1, buf0)
        plsc.subcore_barrier()

        pltpu.sync_copy(
            acc.at[pl.ds(win0, WIN)],
            out_hbm.at[c].at[pl.ds(win0, WIN)],
        )

    return body(src, idx3)


def _combine(partials):
    def body(p_ref, o_ref):
        o_ref[...] = p_ref[0] + p_ref[1]

    blk = 1000
    return pl.pallas_call(
        body,
        grid=(N // blk,),
        in_specs=[pl.BlockSpec((NC, blk, D), lambda i: (0, i, 0))],
        out_specs=pl.BlockSpec((blk, D), lambda i: (i, 0)),
        out_shape=jax.ShapeDtypeStruct((N, D), jnp.float32),
    )(partials)


def kernel(src, index):
    idx3 = index.reshape(NW, NCHUNK, CHUNK)
    partials = _sc_partials(src, idx3)
    return _combine(partials)


# speedup vs baseline: 8.9169x; 1.5047x over previous
import functools

import jax
import jax.numpy as jnp
from jax import lax
from jax.experimental import pallas as pl
from jax.experimental.pallas import tpu as pltpu
from jax.experimental.pallas import tpu_sc as plsc

N = 10000
E = 320000
D = 128

NC = 2
NS = 16
NW = NC * NS

ROWS_PER_TILE = E // NW
CHUNK = 80
NCHUNK = ROWS_PER_TILE // CHUNK
NBUF = 3
WIN = 640
WIN_STRIDE = 624


def _sc_partials(src, idx3):
    mesh = plsc.VectorSubcoreMesh(core_axis_name="c", subcore_axis_name="s")

    @functools.partial(
        pl.kernel,
        mesh=mesh,
        out_type=jax.ShapeDtypeStruct((NC, N, D), jnp.float32),
        scratch_types=[
            pltpu.VMEM_SHARED((N, D), jnp.float32),
            pltpu.VMEM((NCHUNK, CHUNK), jnp.int32),
        ]
        + [pltpu.VMEM((CHUNK, D), jnp.float32) for _ in range(NBUF)]
        + [pltpu.SemaphoreType.DMA for _ in range(2 * NBUF)],
    )
    def body(src_hbm, idx_hbm, out_hbm, acc, idx_v, *rest):
        bufs = rest[:NBUF]
        fsems = rest[NBUF:2 * NBUF]
        ssems = rest[2 * NBUF:]
        c = lax.axis_index("c")
        s = lax.axis_index("s")
        wid = c * NS + s
        row0 = wid * ROWS_PER_TILE
        win0 = pl.multiple_of(s * WIN_STRIDE, 8)

        zeros16 = jnp.zeros((16,), jnp.float32)

        def zero_row(r, _):
            for k in range(D // 16):
                bufs[0][r, pl.ds(k * 16, 16)] = zeros16
            return 0

        lax.fori_loop(0, CHUNK, zero_row, 0)
        for z in range(WIN // CHUNK):
            pltpu.sync_copy(
                bufs[0], acc.at[pl.ds(pl.multiple_of(win0 + z * CHUNK, 8), CHUNK)]
            )
        plsc.subcore_barrier()

        pltpu.sync_copy(idx_hbm.at[wid], idx_v)

        def fill_start(j, b):
            src_off = pl.multiple_of(row0 + j * CHUNK, 8)
            pltpu.async_copy(src_hbm.at[pl.ds(src_off, CHUNK)], bufs[b], fsems[b])

        def fill_wait(b):
            pltpu.make_async_copy(
                src_hbm.at[pl.ds(0, CHUNK)], bufs[b], fsems[b]
            ).wait()

        def scat_start(j, b):
            pltpu.async_copy(bufs[b], acc.at[idx_v.at[j]], ssems[b])

        def scat_wait(j, b):
            pltpu.make_async_copy(bufs[b], acc.at[idx_v.at[j]], ssems[b]).wait()

        fill_start(0, 0)
        fill_start(1, 1)
        fill_wait(0)
        scat_start(0, 0)
        fill_start(2, 2)
        fill_wait(1)
        scat_start(1, 1)
        scat_wait(0, 0)
        fill_start(3, 0)
        fill_wait(2)
        scat_start(2, 2)
        scat_wait(1, 1)
        fill_start(4, 1)

        def tri_step(g, _):
            for u in range(3):
                j = 3 * g + 3 + u
                b = u
                fill_wait(b)
                scat_start(j, b)
                scat_wait(j - 1, (b + 2) % NBUF)
                fill_start(j + 2, (b + 2) % NBUF)
            return 0

        lax.fori_loop(0, (NCHUNK - 5) // 3, tri_step, 0)

        fill_wait(0)
        scat_start(123, 0)
        scat_wait(122, 2)
        fill_wait(1)
        scat_start(124, 1)
        scat_wait(123, 0)
        scat_wait(124, 1)
        plsc.subcore_barrier()

        pltpu.sync_copy(
            acc.at[pl.ds(win0, WIN)],
            out_hbm.at[c].at[pl.ds(win0, WIN)],
        )

    return body(src, idx3)


def _combine(partials):
    def body(p_ref, o_ref):
        o_ref[...] = p_ref[0] + p_ref[1]

    blk = 1000
    return pl.pallas_call(
        body,
        grid=(N // blk,),
        in_specs=[pl.BlockSpec((NC, blk, D), lambda i: (0, i, 0))],
        out_specs=pl.BlockSpec((blk, D), lambda i: (i, 0)),
        out_shape=jax.ShapeDtypeStruct((N, D), jnp.float32),
    )(partials)


def kernel(src, index):
    idx3 = index.reshape(NW, NCHUNK, CHUNK)
    partials = _sc_partials(src, idx3)
    return _combine(partials)
